# baseline (speedup 1.0000x reference)
import functools
import math

import jax
import jax.numpy as jnp
import numpy as np
from jax import lax
from jax.experimental import pallas as pl
from jax.experimental.pallas import tpu as pltpu
from jax.experimental.pallas import tpu_sc as plsc

DT = 128
H = 8
DH = 16
NG = 64
NC = 2
NS = 16
NW = NC * NS
C = 128
L = 16


def _leaky(x):
    return jnp.where(x > 0, x, 0.01 * x)


def _full(shape):
    return pl.BlockSpec(shape, lambda i: tuple(0 for _ in shape))


def _qkv_body(ns, pos4, wq, bq, wk, bk, wv, bv, qt, kt, v):
    x = ns[...]
    pp = pos4[...]
    zp = jnp.zeros((x.shape[0], 124), jnp.float32)
    q = jnp.dot(x, wq[...], preferred_element_type=jnp.float32) + bq[...]
    qt[...] = jnp.concatenate([q, pp, zp], axis=1)
    k = jnp.dot(x, wk[...], preferred_element_type=jnp.float32) + bk[...]
    kt[...] = jnp.concatenate([k, pp, zp], axis=1)
    v[...] = jnp.dot(x, wv[...], preferred_element_type=jnp.float32) + bv[...]


def _qkv(node_s, pos4, p):
    N = node_s.shape[0]
    BN = 2000
    spec_n = pl.BlockSpec((BN, DT), lambda i: (i, 0))
    spec_t = pl.BlockSpec((BN, 2 * DT), lambda i: (i, 0))
    return pl.pallas_call(
        _qkv_body,
        grid=(N // BN,),
        in_specs=[spec_n, pl.BlockSpec((BN, 4), lambda i: (i, 0)),
                  _full((DT, DT)), _full((1, DT)), _full((DT, DT)),
                  _full((1, DT)), _full((DT, DT)), _full((1, DT))],
        out_specs=[spec_t, spec_t, spec_n],
        out_shape=[
            jax.ShapeDtypeStruct((N, 2 * DT), jnp.float32),
            jax.ShapeDtypeStruct((N, 2 * DT), jnp.float32),
            jax.ShapeDtypeStruct((N, DT), jnp.float32),
        ],
    )(node_s, pos4, p["q_W"], p["q_b"].reshape(1, DT), p["k_W"],
      p["k_b"].reshape(1, DT), p["v_W"], p["v_b"].reshape(1, DT))


def _sc_gather(qt, kt, srci, dsti):
    E = srci.shape[0]
    n_chunks = E // C
    n_t = (n_chunks + NW - 1) // NW
    mesh = plsc.VectorSubcoreMesh(core_axis_name="c", subcore_axis_name="s")

    @functools.partial(
        pl.kernel,
        out_type=[
            jax.ShapeDtypeStruct((E, DT), jnp.float32),
            jax.ShapeDtypeStruct((E, 16), jnp.float32),
        ],
        mesh=mesh,
        scratch_types=[
            pltpu.VMEM((C,), jnp.int32),
            pltpu.VMEM((C,), jnp.int32),
            pltpu.VMEM((C, 2 * DT), jnp.float32),
            pltpu.VMEM((C, 2 * DT), jnp.float32),
            pltpu.VMEM((C, DT), jnp.float32),
            pltpu.VMEM((C, 16), jnp.float32),
            pltpu.SemaphoreType.DMA,
            pltpu.SemaphoreType.DMA,
        ],
    )
    def sca(q_h, k_h, src_h, dst_h, qk_h, dx_h,
            sbuf, dbuf, qbuf, kbuf, obuf, dxbuf, s1, s2):
        w = lax.axis_index("c") * NS + lax.axis_index("s")

        def chunk_body(t, carry):
            ch = w + t * NW

            @pl.when(ch < n_chunks)
            def _():
                base = ch * C
                pltpu.sync_copy(src_h.at[pl.ds(base, C)], sbuf)
                pltpu.sync_copy(dst_h.at[pl.ds(base, C)], dbuf)
                cq = pltpu.async_copy(q_h.at[sbuf], qbuf, s1)
                ck = pltpu.async_copy(k_h.at[dbuf], kbuf, s2)
                cq.wait()
                ck.wait()

                def e_body(e, carry2):
                    for h in range(H):
                        sl = pl.ds(h * DH, DH)
                        obuf[e, sl] = qbuf[e, sl] * kbuf[e, sl]
                    sl = pl.ds(DT, L)
                    dxbuf[e, :] = qbuf[e, sl] - kbuf[e, sl]
                    return carry2

                lax.fori_loop(0, C, e_body, 0)
                pltpu.sync_copy(obuf, qk_h.at[pl.ds(base, C)])
                pltpu.sync_copy(dxbuf, dx_h.at[pl.ds(base, C)])

            return carry

        lax.fori_loop(0, n_t, chunk_body, 0)

    return sca(qt, kt, srci, dsti)


def _edge_body(es, qk, dxr, mk,
               m1a, m1d, m1b, m2w, m2b, ew, eb, a1, b1, a2, b2, wh, pmat,
               aux, enew):
    qkv = qk[...]
    d = dxr[...][:, :3]
    t = d + 1e-6
    dist = jnp.sqrt(jnp.sum(t * t, axis=1, keepdims=True))
    h = (jnp.dot(es[...], m1a[...], preferred_element_type=jnp.float32)
         + jnp.dot(dist * 0.1, m1d[...], preferred_element_type=jnp.float32)
         + m1b[...])
    m = jnp.dot(_leaky(h), m2w[...], preferred_element_type=jnp.float32) + m2b[...]
    a = qkv * m * (1.0 / math.sqrt(DH))
    s = jnp.dot(jnp.abs(a), pmat[...], preferred_element_type=jnp.float32)
    e = jnp.exp(s)
    enew[...] = jnp.dot(a, ew[...], preferred_element_type=jnp.float32) + eb[...]
    ab = _leaky(jnp.dot(a, a1[...], preferred_element_type=jnp.float32) + b1[...])
    att = jnp.dot(ab, a2[...], preferred_element_type=jnp.float32) + b2[...]
    scale = jnp.dot(att, wh[...], preferred_element_type=jnp.float32)
    nrm = jnp.sqrt(jnp.sum(d * d, axis=1, keepdims=True))
    dxn = d / (nrm + 1e-6)
    contrib = dxn * scale * mk[...]
    z5 = jnp.zeros((es.shape[0], 5), jnp.float32)
    aux[...] = jnp.concatenate([e, contrib, z5], axis=1)


def _edges(edge_s, qk, dxr, maskcol, p, consts):
    E = edge_s.shape[0]
    BE = 4000
    s16 = pl.BlockSpec((BE, 16), lambda i: (i, 0))
    s128 = pl.BlockSpec((BE, DT), lambda i: (i, 0))
    s1 = pl.BlockSpec((BE, 1), lambda i: (i, 0))
    a1, a2, b1r, pmat = consts
    return pl.pallas_call(
        _edge_body,
        grid=(E // BE,),
        in_specs=[s16, s128, s16, s1,
                  _full((16, DT)), _full((1, DT)), _full((1, DT)),
                  _full((DT, DT)), _full((1, DT)),
                  _full((DT, 16)), _full((1, 16)),
                  _full((DT, 64)), _full((1, 64)),
                  _full((64, 8)), _full((1, 8)),
                  _full((8, 1)), _full((DT, 8))],
        out_specs=[s16, s16],
        out_shape=[
            jax.ShapeDtypeStruct((E, 16), jnp.float32),
            jax.ShapeDtypeStruct((E, 16), jnp.float32),
        ],
    )(edge_s, qk, dxr, maskcol,
      p["m1_W"][:16], p["m1_W"][16:17], p["m1_b"].reshape(1, DT),
      p["m2_W"], p["m2_b"].reshape(1, DT),
      p["e_W"], p["e_b"].reshape(1, 16),
      a1, b1r, a2,
      jnp.full((1, 8), p["a2d2_b"][0], jnp.float32),
      p["wh_W"], pmat)


BNN = 2000


def _stats_update(i, ob, xv, ones, sx, sxx, sc):
    f32 = jnp.float32
    dn = (((0,), (0,)), ((), ()))
    px = lax.dot_general(ob, xv, dn, preferred_element_type=f32)
    pxx = lax.dot_general(ob, xv * xv, dn, preferred_element_type=f32)
    pc = lax.dot_general(ob, ones, dn, preferred_element_type=f32)

    @pl.when(i == 0)
    def _():
        sx[...] = px
        sxx[...] = pxx
        sc[...] = pc

    @pl.when(i > 0)
    def _():
        sx[...] += px
        sxx[...] += pxx
        sc[...] += pc


def _gn_affine(sx, sxx, sc, wv, bv, msv):
    cnt = jnp.maximum(sc[...], 1.0)
    mean = sx[...] / cnt
    var = sxx[...] / cnt - mean * mean * msv[...] * (2.0 - msv[...])
    std = jnp.sqrt(var + 1e-5)
    ga = wv[...] / std
    gb_ = bv[...] - ga * msv[...] * mean
    return ga, gb_


def _n3a_body(acc, ns, o, pos4,
              m2fw, m2fb, g1, g2, g3, gb, qmat,
              g_out, x1_out, pos_out, sx, sxx, sc):
    f32 = jnp.float32
    i = pl.program_id(0)
    accv = acc[...]
    u = accv[:, :DT]
    zc = accv[:, DT:DT + 16]
    zexp = jnp.dot(zc[:, :H], qmat[...], preferred_element_type=f32) + 1e-16
    agg = u / zexp
    nn = jnp.dot(agg, m2fw[...], preferred_element_type=f32) + m2fb[...]
    x = ns[...]
    gpre = (jnp.dot(nn, g1[...], preferred_element_type=f32)
            + jnp.dot(x, g2[...], preferred_element_type=f32)
            + jnp.dot(nn - x, g3[...], preferred_element_type=f32) + gb[...])
    g = jax.nn.sigmoid(gpre)
    g_out[...] = g
    x1 = g * nn + x
    x1_out[...] = x1
    pos_out[...] = pos4[...] + zc[:, H:H + 4]
    ones = jnp.ones((x.shape[0], 1), f32)
    _stats_update(i, o[...], x1, ones, sx, sxx, sc)


def _n3b_body(x1_in, g_in, o, sx1, sxx1, sc1,
              gn1w, gn1b, gn1m, f1w, f1b, f2w, f2b,
              x2_out, sx2, sxx2, sc2):
    f32 = jnp.float32
    i = pl.program_id(0)
    ga, gb_ = _gn_affine(sx1, sxx1, sc1, gn1w, gn1b, gn1m)
    ob = o[...]
    x1n = (jnp.dot(ob, ga, preferred_element_type=f32) * x1_in[...]
           + jnp.dot(ob, gb_, preferred_element_type=f32))
    fin = (jnp.dot(_leaky(jnp.dot(x1n, f1w[...], preferred_element_type=f32) + f1b[...]),
                   f2w[...], preferred_element_type=f32) + f2b[...])
    x2 = g_in[...] * fin + x1n
    x2_out[...] = x2
    ones = jnp.ones((x2.shape[0], 1), f32)
    _stats_update(i, ob, x2, ones, sx2, sxx2, sc2)


def _n3c_body(x2_in, o, sx2, sxx2, sc2, gn2w, gn2b, gn2m, node_out):
    f32 = jnp.float32
    ga, gb_ = _gn_affine(sx2, sxx2, sc2, gn2w, gn2b, gn2m)
    ob = o[...]
    node_out[...] = (jnp.dot(ob, ga, preferred_element_type=f32) * x2_in[...]
                     + jnp.dot(ob, gb_, preferred_element_type=f32))


def _nodes(acc, node_s, onehot, pos4, p, qmat):
    N = node_s.shape[0]
    rv = lambda a: a.reshape(1, -1)
    grid = (N // BNN,)
    sn = pl.BlockSpec((BNN, DT), lambda i: (i, 0))
    sacc = pl.BlockSpec((BNN, DT + 16), lambda i: (i, 0))
    s4 = pl.BlockSpec((BNN, 4), lambda i: (i, 0))
    so = pl.BlockSpec((BNN, NG), lambda i: (i, 0))
    sstat = _full((NG, DT))
    scnt = _full((NG, 1))
    stat_shape = [
        jax.ShapeDtypeStruct((NG, DT), jnp.float32),
        jax.ShapeDtypeStruct((NG, DT), jnp.float32),
        jax.ShapeDtypeStruct((NG, 1), jnp.float32),
    ]

    g, x1, pos_out, sx1, sxx1, sc1 = pl.pallas_call(
        _n3a_body,
        grid=grid,
        in_specs=[sacc, sn, so, s4,
                  _full((DT, DT)), _full((1, DT)), _full((DT, DT)),
                  _full((DT, DT)), _full((DT, DT)), _full((1, DT)),
                  _full((8, DT))],
        out_specs=[sn, sn, s4, sstat, sstat, scnt],
        out_shape=[
            jax.ShapeDtypeStruct((N, DT), jnp.float32),
            jax.ShapeDtypeStruct((N, DT), jnp.float32),
            jax.ShapeDtypeStruct((N, 4), jnp.float32),
        ] + stat_shape,
    )(acc, node_s, onehot, pos4,
      p["m2f_W"], rv(p["m2f_b"]),
      p["gate_W"][:DT], p["gate_W"][DT:2 * DT], p["gate_W"][2 * DT:],
      rv(p["gate_b"]), qmat)

    x2, sx2, sxx2, sc2 = pl.pallas_call(
        _n3b_body,
        grid=grid,
        in_specs=[sn, sn, so, sstat, sstat, scnt,
                  _full((1, DT)), _full((1, DT)), _full((1, DT)),
                  _full((DT, DT)), _full((1, DT)), _full((DT, DT)),
                  _full((1, DT))],
        out_specs=[sn, sstat, sstat, scnt],
        out_shape=[jax.ShapeDtypeStruct((N, DT), jnp.float32)] + stat_shape,
    )(x1, g, onehot, sx1, sxx1, sc1,
      rv(p["gn1_w"]), rv(p["gn1_b"]), rv(p["gn1_ms"]),
      p["fin1_W"], rv(p["fin1_b"]), p["fin2_W"], rv(p["fin2_b"]))

    node_out = pl.pallas_call(
        _n3c_body,
        grid=grid,
        in_specs=[sn, so, sstat, sstat, scnt,
                  _full((1, DT)), _full((1, DT)), _full((1, DT))],
        out_specs=sn,
        out_shape=jax.ShapeDtypeStruct((N, DT), jnp.float32),
    )(x2, onehot, sx2, sxx2, sc2,
      rv(p["gn2_w"]), rv(p["gn2_b"]), rv(p["gn2_ms"]))
    return node_out, pos_out


def kernel(node_s, edge_s, edge_index, total_pos, pro_nodes, batch, params):
    p = params
    src, dst = edge_index[0], edge_index[1]

    di = np.arange(DT)
    pmat = jnp.asarray((di[:, None] // DH == np.arange(H)[None, :]).astype(np.float32))
    qmat = pmat.T * 1.0
    a1 = jnp.asarray(
        np.kron(np.eye(H, dtype=np.float32), np.ones((DH, DH // 2), np.float32))
    ) * jnp.tile(p["a2d1_W"], (H, H))
    b1r = jnp.tile(p["a2d1_b"], H).reshape(1, H * (DH // 2))
    a2 = jnp.asarray(
        np.kron(np.eye(H, dtype=np.float32), np.ones((DH // 2, 1), np.float32))
    ) * jnp.tile(p["a2d2_W"], (H, H))

    pos4 = jnp.pad(total_pos, ((0, 0), (0, 1)))
    qt, kt, v_ = _qkv(node_s, pos4, p)

    qk, dxr = _sc_gather(qt, kt, src, dst)

    maskcol = (src >= pro_nodes).astype(jnp.float32)[:, None]
    aux, enew = _edges(edge_s, qk, dxr, maskcol, p, (a1, a2, b1r, pmat))

    w128 = jnp.repeat(aux[:, :8], 16, axis=1)
    y144 = jnp.concatenate([v_[dst] * w128, aux], axis=1)
    acc = jax.ops.segment_sum(y144, src, num_segments=node_s.shape[0])

    onehot = (batch[:, None] == jnp.arange(NG)[None, :]).astype(jnp.float32)
    node_out, pos_out4 = _nodes(acc, node_s, onehot, pos4, p, qmat)
    return node_out, enew, edge_index, pos_out4[:, :3]


# speedup vs baseline: 1.1483x; 1.1483x over previous
import functools
import math

import jax
import jax.numpy as jnp
import numpy as np
from jax import lax
from jax.experimental import pallas as pl
from jax.experimental.pallas import tpu as pltpu
from jax.experimental.pallas import tpu_sc as plsc

DT = 128
H = 8
DH = 16
NG = 64
NC = 2
NS = 16
NW = NC * NS
C = 128
L = 16


def _leaky(x):
    return jnp.where(x > 0, x, 0.01 * x)


def _full(shape):
    return pl.BlockSpec(shape, lambda i: tuple(0 for _ in shape))


def _qkv_body(ns, pos4, wq, bq, wk, bk, wv, bv, qt, kt, v):
    x = ns[...]
    pp = pos4[...]
    zp = jnp.zeros((x.shape[0], 124), jnp.float32)
    q = jnp.dot(x, wq[...], preferred_element_type=jnp.float32) + bq[...]
    qt[...] = jnp.concatenate([q, pp, zp], axis=1)
    k = jnp.dot(x, wk[...], preferred_element_type=jnp.float32) + bk[...]
    kt[...] = jnp.concatenate([k, pp, zp], axis=1)
    v[...] = jnp.dot(x, wv[...], preferred_element_type=jnp.float32) + bv[...]


def _qkv(node_s, pos4, p):
    N = node_s.shape[0]
    BN = 2000
    spec_n = pl.BlockSpec((BN, DT), lambda i: (i, 0))
    spec_t = pl.BlockSpec((BN, 2 * DT), lambda i: (i, 0))
    return pl.pallas_call(
        _qkv_body,
        grid=(N // BN,),
        in_specs=[spec_n, pl.BlockSpec((BN, 4), lambda i: (i, 0)),
                  _full((DT, DT)), _full((1, DT)), _full((DT, DT)),
                  _full((1, DT)), _full((DT, DT)), _full((1, DT))],
        out_specs=[spec_t, spec_t, spec_n],
        out_shape=[
            jax.ShapeDtypeStruct((N, 2 * DT), jnp.float32),
            jax.ShapeDtypeStruct((N, 2 * DT), jnp.float32),
            jax.ShapeDtypeStruct((N, DT), jnp.float32),
        ],
    )(node_s, pos4, p["q_W"], p["q_b"].reshape(1, DT), p["k_W"],
      p["k_b"].reshape(1, DT), p["v_W"], p["v_b"].reshape(1, DT))


def _sc_gather(qt, kt, srci, dsti):
    E = srci.shape[0]
    n_chunks = E // C
    n_t = (n_chunks + NW - 1) // NW
    mesh = plsc.VectorSubcoreMesh(core_axis_name="c", subcore_axis_name="s")

    @functools.partial(
        pl.kernel,
        out_type=[
            jax.ShapeDtypeStruct((E, DT), jnp.float32),
            jax.ShapeDtypeStruct((E, 16), jnp.float32),
        ],
        mesh=mesh,
        scratch_types=[
            pltpu.VMEM((C,), jnp.int32),
            pltpu.VMEM((C,), jnp.int32),
            pltpu.VMEM((C, 2 * DT), jnp.float32),
            pltpu.VMEM((C, 2 * DT), jnp.float32),
            pltpu.VMEM((C, DT), jnp.float32),
            pltpu.VMEM((C, 16), jnp.float32),
            pltpu.SemaphoreType.DMA,
            pltpu.SemaphoreType.DMA,
        ],
    )
    def sca(q_h, k_h, src_h, dst_h, qk_h, dx_h,
            sbuf, dbuf, qbuf, kbuf, obuf, dxbuf, s1, s2):
        w = lax.axis_index("c") * NS + lax.axis_index("s")

        def chunk_body(t, carry):
            ch = w + t * NW

            @pl.when(ch < n_chunks)
            def _():
                base = ch * C
                pltpu.sync_copy(src_h.at[pl.ds(base, C)], sbuf)
                pltpu.sync_copy(dst_h.at[pl.ds(base, C)], dbuf)
                cq = pltpu.async_copy(q_h.at[sbuf], qbuf, s1)
                ck = pltpu.async_copy(k_h.at[dbuf], kbuf, s2)
                cq.wait()
                ck.wait()

                def e_body(e, carry2):
                    for h in range(H):
                        sl = pl.ds(h * DH, DH)
                        obuf[e, sl] = qbuf[e, sl] * kbuf[e, sl]
                    sl = pl.ds(DT, L)
                    dxbuf[e, :] = qbuf[e, sl] - kbuf[e, sl]
                    return carry2

                lax.fori_loop(0, C, e_body, 0)
                pltpu.sync_copy(obuf, qk_h.at[pl.ds(base, C)])
                pltpu.sync_copy(dxbuf, dx_h.at[pl.ds(base, C)])

            return carry

        lax.fori_loop(0, n_t, chunk_body, 0)

    return sca(qt, kt, srci, dsti)


def _edge_body(es, qk, dxr, mk,
               m1a, m1d, m1b, m2w, m2b, ew, eb, a1, b1, a2, b2, wh, pmat,
               aux, enew):
    qkv = qk[...]
    d = dxr[...][:, :3]
    t = d + 1e-6
    dist = jnp.sqrt(jnp.sum(t * t, axis=1, keepdims=True))
    h = (jnp.dot(es[...], m1a[...], preferred_element_type=jnp.float32)
         + jnp.dot(dist * 0.1, m1d[...], preferred_element_type=jnp.float32)
         + m1b[...])
    m = jnp.dot(_leaky(h), m2w[...], preferred_element_type=jnp.float32) + m2b[...]
    a = qkv * m * (1.0 / math.sqrt(DH))
    s = jnp.dot(jnp.abs(a), pmat[...], preferred_element_type=jnp.float32)
    e = jnp.exp(s)
    enew[...] = jnp.dot(a, ew[...], preferred_element_type=jnp.float32) + eb[...]
    ab = _leaky(jnp.dot(a, a1[...], preferred_element_type=jnp.float32) + b1[...])
    att = jnp.dot(ab, a2[...], preferred_element_type=jnp.float32) + b2[...]
    scale = jnp.dot(att, wh[...], preferred_element_type=jnp.float32)
    nrm = jnp.sqrt(jnp.sum(d * d, axis=1, keepdims=True))
    dxn = d / (nrm + 1e-6)
    contrib = dxn * scale * mk[...]
    z5 = jnp.zeros((es.shape[0], 5), jnp.float32)
    aux[...] = jnp.concatenate([e, contrib, z5], axis=1)


def _edges(edge_s, qk, dxr, maskcol, p, consts):
    E = edge_s.shape[0]
    BE = 4000
    s16 = pl.BlockSpec((BE, 16), lambda i: (i, 0))
    s128 = pl.BlockSpec((BE, DT), lambda i: (i, 0))
    s1 = pl.BlockSpec((BE, 1), lambda i: (i, 0))
    a1, a2, b1r, pmat = consts
    return pl.pallas_call(
        _edge_body,
        grid=(E // BE,),
        in_specs=[s16, s128, s16, s1,
                  _full((16, DT)), _full((1, DT)), _full((1, DT)),
                  _full((DT, DT)), _full((1, DT)),
                  _full((DT, 16)), _full((1, 16)),
                  _full((DT, 64)), _full((1, 64)),
                  _full((64, 8)), _full((1, 8)),
                  _full((8, 1)), _full((DT, 8))],
        out_specs=[s16, s16],
        out_shape=[
            jax.ShapeDtypeStruct((E, 16), jnp.float32),
            jax.ShapeDtypeStruct((E, 16), jnp.float32),
        ],
    )(edge_s, qk, dxr, maskcol,
      p["m1_W"][:16], p["m1_W"][16:17], p["m1_b"].reshape(1, DT),
      p["m2_W"], p["m2_b"].reshape(1, DT),
      p["e_W"], p["e_b"].reshape(1, 16),
      a1, b1r, a2,
      jnp.full((1, 8), p["a2d2_b"][0], jnp.float32),
      p["wh_W"], pmat)


BNN = 2000


def _stats_update(i, ob, xv, ones, sx, sxx, sc):
    f32 = jnp.float32
    dn = (((0,), (0,)), ((), ()))
    px = lax.dot_general(ob, xv, dn, preferred_element_type=f32)
    pxx = lax.dot_general(ob, xv * xv, dn, preferred_element_type=f32)
    pc = lax.dot_general(ob, ones, dn, preferred_element_type=f32)

    @pl.when(i == 0)
    def _():
        sx[...] = px
        sxx[...] = pxx
        sc[...] = pc

    @pl.when(i > 0)
    def _():
        sx[...] += px
        sxx[...] += pxx
        sc[...] += pc


def _gn_affine(sx, sxx, sc, wv, bv, msv):
    cnt = jnp.maximum(sc[...], 1.0)
    mean = sx[...] / cnt
    var = sxx[...] / cnt - mean * mean * msv[...] * (2.0 - msv[...])
    std = jnp.sqrt(var + 1e-5)
    ga = wv[...] / std
    gb_ = bv[...] - ga * msv[...] * mean
    return ga, gb_


def _n3a_body(u_in, z_in, ns, o, pos4,
              m2fw, m2fb, g1, g2, g3, gb, qmat,
              g_out, x1_out, pos_out, sx, sxx, sc):
    f32 = jnp.float32
    i = pl.program_id(0)
    u = u_in[...]
    zc = z_in[...]
    zexp = jnp.dot(zc[:, :H], qmat[...], preferred_element_type=f32) + 1e-16
    agg = u / zexp
    nn = jnp.dot(agg, m2fw[...], preferred_element_type=f32) + m2fb[...]
    x = ns[...]
    gpre = (jnp.dot(nn, g1[...], preferred_element_type=f32)
            + jnp.dot(x, g2[...], preferred_element_type=f32)
            + jnp.dot(nn - x, g3[...], preferred_element_type=f32) + gb[...])
    g = jax.nn.sigmoid(gpre)
    g_out[...] = g
    x1 = g * nn + x
    x1_out[...] = x1
    pos_out[...] = pos4[...] + zc[:, H:H + 4]
    ones = jnp.ones((x.shape[0], 1), f32)
    _stats_update(i, o[...], x1, ones, sx, sxx, sc)


def _n3b_body(x1_in, g_in, o, sx1, sxx1, sc1,
              gn1w, gn1b, gn1m, f1w, f1b, f2w, f2b,
              x2_out, sx2, sxx2, sc2):
    f32 = jnp.float32
    i = pl.program_id(0)
    ga, gb_ = _gn_affine(sx1, sxx1, sc1, gn1w, gn1b, gn1m)
    ob = o[...]
    x1n = (jnp.dot(ob, ga, preferred_element_type=f32) * x1_in[...]
           + jnp.dot(ob, gb_, preferred_element_type=f32))
    fin = (jnp.dot(_leaky(jnp.dot(x1n, f1w[...], preferred_element_type=f32) + f1b[...]),
                   f2w[...], preferred_element_type=f32) + f2b[...])
    x2 = g_in[...] * fin + x1n
    x2_out[...] = x2
    ones = jnp.ones((x2.shape[0], 1), f32)
    _stats_update(i, ob, x2, ones, sx2, sxx2, sc2)


def _n3c_body(x2_in, o, sx2, sxx2, sc2, gn2w, gn2b, gn2m, node_out):
    f32 = jnp.float32
    ga, gb_ = _gn_affine(sx2, sxx2, sc2, gn2w, gn2b, gn2m)
    ob = o[...]
    node_out[...] = (jnp.dot(ob, ga, preferred_element_type=f32) * x2_in[...]
                     + jnp.dot(ob, gb_, preferred_element_type=f32))


def _nodes(u, z, node_s, onehot, pos4, p, qmat):
    N = node_s.shape[0]
    rv = lambda a: a.reshape(1, -1)
    grid = (N // BNN,)
    sn = pl.BlockSpec((BNN, DT), lambda i: (i, 0))
    s16 = pl.BlockSpec((BNN, 16), lambda i: (i, 0))
    s4 = pl.BlockSpec((BNN, 4), lambda i: (i, 0))
    so = pl.BlockSpec((BNN, NG), lambda i: (i, 0))
    sstat = _full((NG, DT))
    scnt = _full((NG, 1))
    stat_shape = [
        jax.ShapeDtypeStruct((NG, DT), jnp.float32),
        jax.ShapeDtypeStruct((NG, DT), jnp.float32),
        jax.ShapeDtypeStruct((NG, 1), jnp.float32),
    ]

    g, x1, pos_out, sx1, sxx1, sc1 = pl.pallas_call(
        _n3a_body,
        grid=grid,
        in_specs=[sn, s16, sn, so, s4,
                  _full((DT, DT)), _full((1, DT)), _full((DT, DT)),
                  _full((DT, DT)), _full((DT, DT)), _full((1, DT)),
                  _full((8, DT))],
        out_specs=[sn, sn, s4, sstat, sstat, scnt],
        out_shape=[
            jax.ShapeDtypeStruct((N, DT), jnp.float32),
            jax.ShapeDtypeStruct((N, DT), jnp.float32),
            jax.ShapeDtypeStruct((N, 4), jnp.float32),
        ] + stat_shape,
    )(u, z, node_s, onehot, pos4,
      p["m2f_W"], rv(p["m2f_b"]),
      p["gate_W"][:DT], p["gate_W"][DT:2 * DT], p["gate_W"][2 * DT:],
      rv(p["gate_b"]), qmat)

    x2, sx2, sxx2, sc2 = pl.pallas_call(
        _n3b_body,
        grid=grid,
        in_specs=[sn, sn, so, sstat, sstat, scnt,
                  _full((1, DT)), _full((1, DT)), _full((1, DT)),
                  _full((DT, DT)), _full((1, DT)), _full((DT, DT)),
                  _full((1, DT))],
        out_specs=[sn, sstat, sstat, scnt],
        out_shape=[jax.ShapeDtypeStruct((N, DT), jnp.float32)] + stat_shape,
    )(x1, g, onehot, sx1, sxx1, sc1,
      rv(p["gn1_w"]), rv(p["gn1_b"]), rv(p["gn1_ms"]),
      p["fin1_W"], rv(p["fin1_b"]), p["fin2_W"], rv(p["fin2_b"]))

    node_out = pl.pallas_call(
        _n3c_body,
        grid=grid,
        in_specs=[sn, so, sstat, sstat, scnt,
                  _full((1, DT)), _full((1, DT)), _full((1, DT))],
        out_specs=sn,
        out_shape=jax.ShapeDtypeStruct((N, DT), jnp.float32),
    )(x2, onehot, sx2, sxx2, sc2,
      rv(p["gn2_w"]), rv(p["gn2_b"]), rv(p["gn2_ms"]))
    return node_out, pos_out


def kernel(node_s, edge_s, edge_index, total_pos, pro_nodes, batch, params):
    p = params
    src, dst = edge_index[0], edge_index[1]

    di = np.arange(DT)
    pmat = jnp.asarray((di[:, None] // DH == np.arange(H)[None, :]).astype(np.float32))
    qmat = pmat.T * 1.0
    a1 = jnp.asarray(
        np.kron(np.eye(H, dtype=np.float32), np.ones((DH, DH // 2), np.float32))
    ) * jnp.tile(p["a2d1_W"], (H, H))
    b1r = jnp.tile(p["a2d1_b"], H).reshape(1, H * (DH // 2))
    a2 = jnp.asarray(
        np.kron(np.eye(H, dtype=np.float32), np.ones((DH // 2, 1), np.float32))
    ) * jnp.tile(p["a2d2_W"], (H, H))

    pos4 = jnp.pad(total_pos, ((0, 0), (0, 1)))
    qt, kt, v_ = _qkv(node_s, pos4, p)

    qk, dxr = _sc_gather(qt, kt, src, dst)

    maskcol = (src >= pro_nodes).astype(jnp.float32)[:, None]
    aux, enew = _edges(edge_s, qk, dxr, maskcol, p, (a1, a2, b1r, pmat))

    w128 = jnp.repeat(aux[:, :8], 16, axis=1)
    u = jax.ops.segment_sum(v_[dst] * w128, src, num_segments=node_s.shape[0])
    z = jax.ops.segment_sum(aux, src, num_segments=node_s.shape[0])

    onehot = (batch[:, None] == jnp.arange(NG)[None, :]).astype(jnp.float32)
    node_out, pos_out4 = _nodes(u, z, node_s, onehot, pos4, p, qmat)
    return node_out, enew, edge_index, pos_out4[:, :3]
